# baseline (speedup 1.0000x reference)
import functools

import jax
import jax.numpy as jnp
from jax import lax
from jax.experimental import pallas as pl
from jax.experimental.pallas import tpu as pltpu
from jax.experimental.pallas import tpu_sc as plsc

N = 10000
E = 320000
D = 128
NC = 2
NS = 16
EP = 327680
K = 128
CPT = EP // K // NS
CHG = NS * CPT
KE = 64
CPTE = EP // KE // NS
CHGE = NS * CPTE
NB = 4
NBD = 8
ROWS_PER_TILE = (N // NS) // 8 * 8
ROWS_TAIL = N - NS * ROWS_PER_TILE
NPAD = 10240
DEG_PER_TILE = NPAD // NS


def _deg_body(dst_hbm, out_hbm, acc, zbuf, ones_v, didx, sem):
    c = lax.axis_index("c")
    s = lax.axis_index("s")

    def fill(i, _):
        zbuf[pl.ds(i * 16, 16)] = jnp.zeros((16,), jnp.float32)
        ones_v[pl.ds((i % 8) * 16, 16)] = jnp.ones((16,), jnp.float32)
        return 0

    lax.fori_loop(0, DEG_PER_TILE // 16, fill, 0)
    pltpu.sync_copy(zbuf, acc.at[pl.ds(s * DEG_PER_TILE, DEG_PER_TILE)])
    row0 = pl.multiple_of(c * CHG + s * CPT, 8)
    pltpu.sync_copy(dst_hbm.at[pl.ds(row0, CPT)], didx)
    plsc.subcore_barrier()

    G = CPT // NBD

    for b in range(NBD):
        pltpu.async_copy(ones_v, acc.at[didx.at[b]], sem, add=True)

    def grp(g, _):
        base = g * NBD

        @pl.when(g < G - 1)
        def _():
            for b in range(NBD):
                pltpu.async_copy(ones_v, acc.at[didx.at[base + NBD + b]],
                                 sem, add=True)

        for b in range(NBD):
            pltpu.make_async_copy(ones_v, acc.at[didx.at[base + b]],
                                  sem).wait()
        return 0

    lax.fori_loop(0, G, grp, 0)
    plsc.subcore_barrier()
    ooff = pl.multiple_of(c * NPAD + s * DEG_PER_TILE, 8)
    pltpu.sync_copy(acc.at[pl.ds(s * DEG_PER_TILE, DEG_PER_TILE)],
                    out_hbm.at[pl.ds(ooff, DEG_PER_TILE)])


IBLK = 32


def _edge_body(y_hbm, src_hbm, dst_hbm, out_hbm, acc,
               b0, b1, b2, b3, sidxb, didxb, sem_g, sem_s, sem_i):
    bufs = [b0, b1, b2, b3]
    c = lax.axis_index("c")
    s = lax.axis_index("s")

    roff = pl.multiple_of(s * ROWS_PER_TILE, 8)
    groff = pl.multiple_of(c * N + s * ROWS_PER_TILE, 8)
    pltpu.sync_copy(y_hbm.at[pl.ds(groff, ROWS_PER_TILE)],
                    acc.at[pl.ds(roff, ROWS_PER_TILE)])
    tail = NS * ROWS_PER_TILE

    @pl.when(s == 0)
    def _():
        pltpu.sync_copy(y_hbm.at[pl.ds(pl.multiple_of(c * N + tail, 8),
                                       ROWS_TAIL)],
                        acc.at[pl.ds(tail, ROWS_TAIL)])

    crow0 = pl.multiple_of(c * CHGE + s * CPTE, 8)

    pltpu.sync_copy(src_hbm.at[pl.ds(crow0, IBLK)], sidxb.at[pl.ds(0, IBLK)])
    pltpu.sync_copy(dst_hbm.at[pl.ds(crow0, IBLK)], didxb.at[pl.ds(0, IBLK)])
    pltpu.async_copy(src_hbm.at[pl.ds(crow0 + IBLK, IBLK)],
                     sidxb.at[pl.ds(IBLK, IBLK)], sem_i)
    pltpu.async_copy(dst_hbm.at[pl.ds(crow0 + IBLK, IBLK)],
                     didxb.at[pl.ds(IBLK, IBLK)], sem_i)
    plsc.subcore_barrier()

    G = CPTE // NB
    GPB = IBLK // NB

    def grp(g, _):
        i0 = g * NB

        @pl.when(g > 0)
        def _():
            for b in range(NB):
                pltpu.make_async_copy(bufs[b], acc.at[didxb.at[0]],
                                      sem_s.at[b]).wait()

        @pl.when(jnp.logical_and(g % GPB == 0, g > 0))
        def _():
            pltpu.make_async_copy(src_hbm.at[pl.ds(crow0, IBLK)],
                                  sidxb.at[pl.ds(0, IBLK)], sem_i).wait()
            pltpu.make_async_copy(dst_hbm.at[pl.ds(crow0, IBLK)],
                                  didxb.at[pl.ds(0, IBLK)], sem_i).wait()

        for b in range(NB):
            row = (i0 + b) % (2 * IBLK)
            pltpu.async_copy(y_hbm.at[sidxb.at[row]], bufs[b], sem_g.at[b])

        @pl.when(jnp.logical_and(g % GPB == 0,
                                 jnp.logical_and(g > 0, g < G - GPB)))
        def _():
            nxt = pl.multiple_of(i0 + IBLK, IBLK)
            dst_off = pl.multiple_of(nxt % (2 * IBLK), IBLK)
            pltpu.async_copy(src_hbm.at[pl.ds(pl.multiple_of(crow0 + nxt, 8),
                                              IBLK)],
                             sidxb.at[pl.ds(dst_off, IBLK)], sem_i)
            pltpu.async_copy(dst_hbm.at[pl.ds(pl.multiple_of(crow0 + nxt, 8),
                                              IBLK)],
                             didxb.at[pl.ds(dst_off, IBLK)], sem_i)

        for b in range(NB):
            row = (i0 + b) % (2 * IBLK)
            pltpu.make_async_copy(y_hbm.at[sidxb.at[row]], bufs[b],
                                  sem_g.at[b]).wait()
            pltpu.async_copy(bufs[b], acc.at[didxb.at[row]], sem_s.at[b],
                             add=True)
        return 0

    lax.fori_loop(0, G, grp, 0)
    for b in range(NB):
        pltpu.make_async_copy(bufs[b], acc.at[didxb.at[0]],
                              sem_s.at[b]).wait()
    plsc.subcore_barrier()

    pltpu.sync_copy(acc.at[pl.ds(roff, ROWS_PER_TILE)],
                    out_hbm.at[pl.ds(groff, ROWS_PER_TILE)])

    @pl.when(s == 0)
    def _():
        pltpu.sync_copy(acc.at[pl.ds(tail, ROWS_TAIL)],
                        out_hbm.at[pl.ds(pl.multiple_of(c * N + tail, 8),
                                         ROWS_TAIL)])


@functools.lru_cache(maxsize=None)
def _sc_kernels():
    mesh = plsc.VectorSubcoreMesh(core_axis_name="c", subcore_axis_name="s",
                                  num_cores=NC, num_subcores=NS)
    deg_k = pl.kernel(
        _deg_body,
        mesh=mesh,
        out_type=jax.ShapeDtypeStruct((2 * NPAD,), jnp.float32),
        scratch_types=[
            pltpu.VMEM_SHARED((NPAD,), jnp.float32),
            pltpu.VMEM((DEG_PER_TILE,), jnp.float32),
            pltpu.VMEM((K,), jnp.float32),
            pltpu.VMEM((CPT, K), jnp.int32),
            pltpu.SemaphoreType.DMA,
        ],
    )
    edge_k = pl.kernel(
        _edge_body,
        mesh=mesh,
        out_type=jax.ShapeDtypeStruct((2 * N, D), jnp.float32),
        scratch_types=(
            [pltpu.VMEM_SHARED((N + 8, D), jnp.float32)]
            + [pltpu.VMEM((KE, D), jnp.float32) for _ in range(NB)]
            + [pltpu.VMEM((2 * IBLK, KE), jnp.int32) for _ in range(2)]
            + [pltpu.SemaphoreType.DMA((NB,)),
               pltpu.SemaphoreType.DMA((NB,)),
               pltpu.SemaphoreType.DMA]
        ),
    )
    return deg_k, edge_k


_RB = 1000
_GRID = 2 * N // _RB


def _scale_body(x_ref, deg_ref, o_ref):
    dinv = lax.rsqrt(deg_ref[...] + 1.0)
    o_ref[...] = x_ref[...] * dinv


def _scale(x, deg):
    return pl.pallas_call(
        _scale_body,
        grid=(_GRID,),
        in_specs=[
            pl.BlockSpec((_RB, D), lambda i: (i, 0)),
            pl.BlockSpec((_RB, 1), lambda i: (i, 0)),
        ],
        out_specs=pl.BlockSpec((_RB, D), lambda i: (i, 0)),
        out_shape=jax.ShapeDtypeStruct((2 * N, D), jnp.float32),
    )(x, deg)


def _conv_relu_body(z_ref, deg_ref, w_ref, b_ref, o_ref):
    dinv = lax.rsqrt(deg_ref[...] + 1.0)
    h = jnp.dot(z_ref[...] * dinv, w_ref[0],
                preferred_element_type=jnp.float32) + b_ref[0]
    o_ref[...] = jnp.maximum(h, 0.0) * dinv


def _conv_relu_scale(z, deg, w2, b2):
    return pl.pallas_call(
        _conv_relu_body,
        grid=(_GRID,),
        in_specs=[
            pl.BlockSpec((_RB, D), lambda i: (i, 0)),
            pl.BlockSpec((_RB, 1), lambda i: (i, 0)),
            pl.BlockSpec((1, D, D), lambda i: (i // (_GRID // 2), 0, 0)),
            pl.BlockSpec((1, 1, D), lambda i: (i // (_GRID // 2), 0, 0)),
        ],
        out_specs=pl.BlockSpec((_RB, D), lambda i: (i, 0)),
        out_shape=jax.ShapeDtypeStruct((2 * N, D), jnp.float32),
    )(z, deg, w2, b2)


def _head_body(z_ref, deg_ref, w_ref, b_ref, wm1, bm1, wm2, bm2, wm3, bm3,
               o_ref):
    dinv = lax.rsqrt(deg_ref[...] + 1.0)
    emb = jnp.dot(z_ref[...] * dinv, w_ref[0],
                  preferred_element_type=jnp.float32) + b_ref[0]
    h = jnp.maximum(jnp.dot(emb, wm1[...],
                            preferred_element_type=jnp.float32) + bm1[...], 0.0)
    h = jnp.maximum(jnp.dot(h, wm2[...],
                            preferred_element_type=jnp.float32) + bm2[...], 0.0)
    o_ref[...] = jnp.dot(h, wm3[...],
                         preferred_element_type=jnp.float32) + bm3[...]


def _head(z, deg, w2, b2, wm1, bm1, wm2, bm2, wm3, bm3):
    full = lambda shape: pl.BlockSpec(shape, lambda i: (0,) * len(shape))
    return pl.pallas_call(
        _head_body,
        grid=(_GRID,),
        in_specs=[
            pl.BlockSpec((_RB, D), lambda i: (i, 0)),
            pl.BlockSpec((_RB, 1), lambda i: (i, 0)),
            pl.BlockSpec((1, D, D), lambda i: (i // (_GRID // 2), 0, 0)),
            pl.BlockSpec((1, 1, D), lambda i: (i // (_GRID // 2), 0, 0)),
            full((D, D)), full((1, D)), full((D, D)), full((1, D)),
            full((D, 1)), full((1, 1)),
        ],
        out_specs=pl.BlockSpec((_RB, 1), lambda i: (i, 0)),
        out_shape=jax.ShapeDtypeStruct((2 * N, 1), jnp.float32),
    )(z, deg, w2, b2, wm1, bm1, wm2, bm2, wm3, bm3)


def kernel(p_node_feat, p_edge_index, r_node_feat, r_edge_index,
           Wp1, bp1, Wp2, bp2, Wr1, br1, Wr2, br2,
           Wm1, bm1, Wm2, bm2, Wm3, bm3):
    x = jnp.concatenate([p_node_feat, r_node_feat], axis=0)
    pad0 = jnp.zeros((EP - E,), jnp.int32)
    padN = jnp.full((EP - E,), N, jnp.int32)
    src_p = jnp.concatenate([p_edge_index[0], pad0])
    src_r = jnp.concatenate([r_edge_index[0] + N, pad0])
    dst_p = jnp.concatenate([p_edge_index[1], padN])
    dst_r = jnp.concatenate([r_edge_index[1], padN])
    dstm = jnp.concatenate([dst_p, dst_r]).reshape(2 * CHG, K)
    srcm64 = jnp.concatenate([src_p, src_r]).reshape(2 * CHGE, KE)
    dstm64 = jnp.concatenate([dst_p, dst_r]).reshape(2 * CHGE, KE)

    deg_kernel, edge_kernel = _sc_kernels()
    deg_raw = deg_kernel(dstm)
    deg = jnp.concatenate([deg_raw[:N], deg_raw[NPAD:NPAD + N]])
    deg = deg.reshape(2 * N, 1)

    w1 = jnp.stack([Wp1, Wr1])
    b1 = jnp.stack([bp1, br1]).reshape(2, 1, D)
    w2 = jnp.stack([Wp2, Wr2])
    b2 = jnp.stack([bp2, br2]).reshape(2, 1, D)

    y1 = _scale(x, deg)
    z1 = edge_kernel(y1, srcm64, dstm64)
    y2 = _conv_relu_scale(z1, deg, w1, b1)
    z2 = edge_kernel(y2, srcm64, dstm64)
    return _head(z2, deg, w2, b2,
                 Wm1, bm1.reshape(1, D), Wm2, bm2.reshape(1, D),
                 Wm3, bm3.reshape(1, 1))


# speedup vs baseline: 1.0220x; 1.0220x over previous
import functools

import jax
import jax.numpy as jnp
from jax import lax
from jax.experimental import pallas as pl
from jax.experimental.pallas import tpu as pltpu
from jax.experimental.pallas import tpu_sc as plsc

N = 10000
E = 320000
D = 128
NC = 2
NS = 16
K = 128
CPT = 160
EP = CPT * K * NS
CHG = NS * CPT
NBD = 8
ROWS_PER_TILE = (N // NS) // 8 * 8
ROWS_TAIL = N - NS * ROWS_PER_TILE
NPAD = 10240
DEG_PER_TILE = NPAD // NS


def _deg_body(dst_hbm, out_hbm, acc, zbuf, ones_v, didx, sem):
    c = lax.axis_index("c")
    s = lax.axis_index("s")

    def fill(i, _):
        zbuf[pl.ds(i * 16, 16)] = jnp.zeros((16,), jnp.float32)
        ones_v[pl.ds((i % 8) * 16, 16)] = jnp.ones((16,), jnp.float32)
        return 0

    lax.fori_loop(0, DEG_PER_TILE // 16, fill, 0)
    pltpu.sync_copy(zbuf, acc.at[pl.ds(s * DEG_PER_TILE, DEG_PER_TILE)])
    row0 = pl.multiple_of(c * CHG + s * CPT, 8)
    pltpu.sync_copy(dst_hbm.at[pl.ds(row0, CPT)], didx)
    plsc.subcore_barrier()

    G = CPT // NBD

    for b in range(NBD):
        pltpu.async_copy(ones_v, acc.at[didx.at[b]], sem, add=True)

    def grp(g, _):
        base = g * NBD

        @pl.when(g < G - 1)
        def _():
            for b in range(NBD):
                pltpu.async_copy(ones_v, acc.at[didx.at[base + NBD + b]],
                                 sem, add=True)

        for b in range(NBD):
            pltpu.make_async_copy(ones_v, acc.at[didx.at[base + b]],
                                  sem).wait()
        return 0

    lax.fori_loop(0, G, grp, 0)
    plsc.subcore_barrier()
    ooff = pl.multiple_of(c * NPAD + s * DEG_PER_TILE, 8)
    pltpu.sync_copy(acc.at[pl.ds(s * DEG_PER_TILE, DEG_PER_TILE)],
                    out_hbm.at[pl.ds(ooff, DEG_PER_TILE)])


def _edge_body(y_hbm, sd_hbm, out_hbm, acc,
               b0, b1, b2, i0, i1, i2, i3, i4, i5, sem_g, sem_s):
    bufs = [b0, b1, b2]
    isl = [i0, i1, i2, i3, i4, i5]
    c = lax.axis_index("c")
    s = lax.axis_index("s")

    roff = pl.multiple_of(s * ROWS_PER_TILE, 8)
    groff = pl.multiple_of(c * N + s * ROWS_PER_TILE, 8)
    pltpu.sync_copy(y_hbm.at[pl.ds(groff, ROWS_PER_TILE)],
                    acc.at[pl.ds(roff, ROWS_PER_TILE)])
    tail = NS * ROWS_PER_TILE

    @pl.when(s == 0)
    def _():
        pltpu.sync_copy(y_hbm.at[pl.ds(pl.multiple_of(c * N + tail, 8),
                                       ROWS_TAIL)],
                        acc.at[pl.ds(tail, ROWS_TAIL)])

    crow0 = c * CHG + s * CPT
    plsc.subcore_barrier()

    def stages(i, t, first_two=False, has_next=True):
        b, bn = t % 3, (t + 1) % 3
        if has_next:
            pltpu.sync_copy(sd_hbm.at[crow0 + i + 1], isl[(t + 1) % 6])
        if not first_two:
            pltpu.make_async_copy(bufs[bn], acc.at[isl[t].at[1]],
                                  sem_s.at[bn]).wait()
        pltpu.make_async_copy(y_hbm.at[isl[t].at[0]], bufs[b],
                              sem_g.at[b]).wait()
        pltpu.async_copy(bufs[b], acc.at[isl[t].at[1]], sem_s.at[b],
                         add=True)
        if has_next:
            pltpu.async_copy(y_hbm.at[isl[(t + 1) % 6].at[0]], bufs[bn],
                             sem_g.at[bn])

    pltpu.sync_copy(sd_hbm.at[crow0], isl[0])
    pltpu.async_copy(y_hbm.at[isl[0].at[0]], bufs[0], sem_g.at[0])
    for t in range(6):
        stages(t, t, first_two=(t < 2))

    def grp(g, _):
        base = g * 6
        for t in range(6):
            stages(base + t, t)
        return 0

    lax.fori_loop(1, (CPT - 4) // 6, grp, 0)
    for u in range(4):
        stages(CPT - 4 + u, u, has_next=(u < 3))
    for b in (2, 0):
        pltpu.make_async_copy(bufs[b], acc.at[isl[0].at[1]],
                              sem_s.at[b]).wait()
    plsc.subcore_barrier()

    pltpu.sync_copy(acc.at[pl.ds(roff, ROWS_PER_TILE)],
                    out_hbm.at[pl.ds(groff, ROWS_PER_TILE)])

    @pl.when(s == 0)
    def _():
        pltpu.sync_copy(acc.at[pl.ds(tail, ROWS_TAIL)],
                        out_hbm.at[pl.ds(pl.multiple_of(c * N + tail, 8),
                                         ROWS_TAIL)])


@functools.lru_cache(maxsize=None)
def _sc_kernels():
    mesh = plsc.VectorSubcoreMesh(core_axis_name="c", subcore_axis_name="s",
                                  num_cores=NC, num_subcores=NS)
    deg_k = pl.kernel(
        _deg_body,
        mesh=mesh,
        out_type=jax.ShapeDtypeStruct((2 * NPAD,), jnp.float32),
        scratch_types=[
            pltpu.VMEM_SHARED((NPAD,), jnp.float32),
            pltpu.VMEM((DEG_PER_TILE,), jnp.float32),
            pltpu.VMEM((K,), jnp.float32),
            pltpu.VMEM((CPT, K), jnp.int32),
            pltpu.SemaphoreType.DMA,
        ],
    )
    edge_k = pl.kernel(
        _edge_body,
        mesh=mesh,
        out_type=jax.ShapeDtypeStruct((2 * N, D), jnp.float32),
        scratch_types=(
            [pltpu.VMEM_SHARED((N + 2, D), jnp.float32)]
            + [pltpu.VMEM((K, D), jnp.float32) for _ in range(3)]
            + [pltpu.VMEM((2, K), jnp.int32) for _ in range(6)]
            + [pltpu.SemaphoreType.DMA((3,)),
               pltpu.SemaphoreType.DMA((3,))]
        ),
    )
    return deg_k, edge_k


_RB = 1000
_GRID = 2 * N // _RB


def _scale_body(x_ref, deg_ref, o_ref):
    dinv = lax.rsqrt(deg_ref[...] + 1.0)
    o_ref[...] = x_ref[...] * dinv


def _scale(x, deg):
    return pl.pallas_call(
        _scale_body,
        grid=(_GRID,),
        in_specs=[
            pl.BlockSpec((_RB, D), lambda i: (i, 0)),
            pl.BlockSpec((_RB, 1), lambda i: (i, 0)),
        ],
        out_specs=pl.BlockSpec((_RB, D), lambda i: (i, 0)),
        out_shape=jax.ShapeDtypeStruct((2 * N, D), jnp.float32),
    )(x, deg)


def _conv_relu_body(z_ref, deg_ref, w_ref, b_ref, o_ref):
    dinv = lax.rsqrt(deg_ref[...] + 1.0)
    h = jnp.dot(z_ref[...] * dinv, w_ref[0],
                preferred_element_type=jnp.float32) + b_ref[0]
    o_ref[...] = jnp.maximum(h, 0.0) * dinv


def _conv_relu_scale(z, deg, w2, b2):
    return pl.pallas_call(
        _conv_relu_body,
        grid=(_GRID,),
        in_specs=[
            pl.BlockSpec((_RB, D), lambda i: (i, 0)),
            pl.BlockSpec((_RB, 1), lambda i: (i, 0)),
            pl.BlockSpec((1, D, D), lambda i: (i // (_GRID // 2), 0, 0)),
            pl.BlockSpec((1, 1, D), lambda i: (i // (_GRID // 2), 0, 0)),
        ],
        out_specs=pl.BlockSpec((_RB, D), lambda i: (i, 0)),
        out_shape=jax.ShapeDtypeStruct((2 * N, D), jnp.float32),
    )(z, deg, w2, b2)


def _head_body(z_ref, deg_ref, w_ref, b_ref, wm1, bm1, wm2, bm2, wm3, bm3,
               o_ref):
    dinv = lax.rsqrt(deg_ref[...] + 1.0)
    emb = jnp.dot(z_ref[...] * dinv, w_ref[0],
                  preferred_element_type=jnp.float32) + b_ref[0]
    h = jnp.maximum(jnp.dot(emb, wm1[...],
                            preferred_element_type=jnp.float32) + bm1[...], 0.0)
    h = jnp.maximum(jnp.dot(h, wm2[...],
                            preferred_element_type=jnp.float32) + bm2[...], 0.0)
    o_ref[...] = jnp.dot(h, wm3[...],
                         preferred_element_type=jnp.float32) + bm3[...]


def _head(z, deg, w2, b2, wm1, bm1, wm2, bm2, wm3, bm3):
    full = lambda shape: pl.BlockSpec(shape, lambda i: (0,) * len(shape))
    return pl.pallas_call(
        _head_body,
        grid=(_GRID,),
        in_specs=[
            pl.BlockSpec((_RB, D), lambda i: (i, 0)),
            pl.BlockSpec((_RB, 1), lambda i: (i, 0)),
            pl.BlockSpec((1, D, D), lambda i: (i // (_GRID // 2), 0, 0)),
            pl.BlockSpec((1, 1, D), lambda i: (i // (_GRID // 2), 0, 0)),
            full((D, D)), full((1, D)), full((D, D)), full((1, D)),
            full((D, 1)), full((1, 1)),
        ],
        out_specs=pl.BlockSpec((_RB, 1), lambda i: (i, 0)),
        out_shape=jax.ShapeDtypeStruct((2 * N, 1), jnp.float32),
    )(z, deg, w2, b2, wm1, bm1, wm2, bm2, wm3, bm3)


def kernel(p_node_feat, p_edge_index, r_node_feat, r_edge_index,
           Wp1, bp1, Wp2, bp2, Wr1, br1, Wr2, br2,
           Wm1, bm1, Wm2, bm2, Wm3, bm3):
    x = jnp.concatenate([p_node_feat, r_node_feat], axis=0)
    pad0 = jnp.zeros((EP - E,), jnp.int32)
    padN = jnp.full((EP - E,), N, jnp.int32)
    src_p = jnp.concatenate([p_edge_index[0], pad0])
    src_r = jnp.concatenate([r_edge_index[0] + N, pad0])
    dst_p = jnp.concatenate([p_edge_index[1], padN])
    dst_r = jnp.concatenate([r_edge_index[1], padN])
    dstm = jnp.concatenate([dst_p, dst_r]).reshape(2 * CHG, K)
    sd = jnp.concatenate([
        jnp.stack([src_p.reshape(CHG, K), dst_p.reshape(CHG, K)], axis=1),
        jnp.stack([src_r.reshape(CHG, K), dst_r.reshape(CHG, K)], axis=1),
    ])

    deg_kernel, edge_kernel = _sc_kernels()
    deg_raw = deg_kernel(dstm)
    deg = jnp.concatenate([deg_raw[:N], deg_raw[NPAD:NPAD + N]])
    deg = deg.reshape(2 * N, 1)

    w1 = jnp.stack([Wp1, Wr1])
    b1 = jnp.stack([bp1, br1]).reshape(2, 1, D)
    w2 = jnp.stack([Wp2, Wr2])
    b2 = jnp.stack([bp2, br2]).reshape(2, 1, D)

    y1 = _scale(x, deg)
    z1 = edge_kernel(y1, sd)
    y2 = _conv_relu_scale(z1, deg, w1, b1)
    z2 = edge_kernel(y2, sd)
    return _head(z2, deg, w2, b2,
                 Wm1, bm1.reshape(1, D), Wm2, bm2.reshape(1, D),
                 Wm3, bm3.reshape(1, 1))
